# baseline (speedup 1.0000x reference)
import functools
import math

import jax
import jax.numpy as jnp
from jax import lax
from jax.experimental import pallas as pl
from jax.experimental.pallas import tpu as pltpu
from jax.experimental.pallas import tpu_sc as plsc

LANES = 16
NC, NS = 2, 16
NW = NC * NS
CHUNK = 128
NBUF = 4
OUTW = 128


def _build(n_idx, vocab, d):
    per_w = n_idx // NW
    n_ch = per_w // CHUNK
    n_grp = n_ch // NBUF
    scale = math.sqrt(d)
    n_col = d // LANES
    row_unroll = 4

    mesh = plsc.VectorSubcoreMesh(core_axis_name="c", subcore_axis_name="s")

    @functools.partial(
        pl.kernel,
        mesh=mesh,
        compiler_params=pltpu.CompilerParams(use_tc_tiling_on_sc=False),
        out_type=jax.ShapeDtypeStruct((n_idx, OUTW), jnp.float32),
        scratch_types=[
            pltpu.VMEM((n_ch, CHUNK), jnp.int32),
            *[pltpu.VMEM((CHUNK, d), jnp.float32) for _ in range(NBUF)],
            *[pltpu.VMEM((CHUNK, OUTW), jnp.float32) for _ in range(NBUF)],
            *[pltpu.SemaphoreType.DMA for _ in range(2 * NBUF)],
        ],
    )
    def run(tok_hbm, table_hbm, out_hbm, idx_v, *rest):
        gbufs = rest[0:NBUF]
        sbufs = rest[NBUF:2 * NBUF]
        gsems = rest[2 * NBUF:3 * NBUF]
        ssems = rest[3 * NBUF:4 * NBUF]

        wid = lax.axis_index("s") * NC + lax.axis_index("c")
        base = wid * per_w

        pltpu.sync_copy(tok_hbm.at[wid], idx_v)

        def start_gather(j, b):
            pltpu.async_copy(table_hbm.at[idx_v.at[j]], gbufs[b], gsems[b])

        def wait_gather(b):
            pltpu.make_async_copy(
                table_hbm.at[pl.ds(0, CHUNK)], gbufs[b], gsems[b]).wait()

        def start_store(j, b):
            pltpu.async_copy(
                sbufs[b], out_hbm.at[pl.ds(base + j * CHUNK, CHUNK)], ssems[b])

        def wait_store(b):
            pltpu.make_async_copy(
                sbufs[b], out_hbm.at[pl.ds(0, CHUNK)], ssems[b]).wait()

        def scale_buf(b):
            gb, sb = gbufs[b], sbufs[b]

            def body(i, _):
                r0 = i * row_unroll
                for u in range(row_unroll):
                    for c in range(n_col):
                        sb[r0 + u, pl.ds(c * LANES, LANES)] = (
                            gb[r0 + u, pl.ds(c * LANES, LANES)] * scale)
                return 0

            lax.fori_loop(0, CHUNK // row_unroll, body, 0)

        for b in range(NBUF):
            start_gather(b, b)

        for b in range(NBUF):
            wait_gather(b)
            scale_buf(b)
            start_store(b, b)
            start_gather(b + NBUF, b)

        def group_body(g, _):
            for b in range(NBUF):
                j = g * NBUF + b
                wait_gather(b)
                wait_store(b)
                scale_buf(b)
                start_store(j, b)
                start_gather(j + NBUF, b)
            return 0

        lax.fori_loop(1, n_grp - 1, group_body, 0)

        for b in range(NBUF):
            j = (n_grp - 1) * NBUF + b
            wait_gather(b)
            wait_store(b)
            scale_buf(b)
            start_store(j, b)

        for b in range(NBUF):
            wait_store(b)

    return run


def kernel(tokens, embedding):
    b, l = tokens.shape
    vocab, d = embedding.shape
    n_idx = b * l
    assert n_idx % (NW * CHUNK * NBUF) == 0 and d % LANES == 0
    tok = tokens.reshape(NW, n_idx // (NW * CHUNK), CHUNK).astype(jnp.int32)
    out = _build(n_idx, vocab, d)(tok, embedding)
    return out[:, :d].reshape(b, l, d)


# speedup vs baseline: 1.1645x; 1.1645x over previous
import functools
import math

import jax
import jax.numpy as jnp
from jax import lax
from jax.experimental import pallas as pl
from jax.experimental.layout import Format, Layout, with_layout_constraint
from jax.experimental.pallas import tpu as pltpu
from jax.experimental.pallas import tpu_sc as plsc

LANES = 16
NC, NS = 2, 16
NW = NC * NS
CHUNK = 128
NBUF = 4
OUTW = 128


def _build(n_idx, vocab, d):
    per_w = n_idx // NW
    n_ch = per_w // CHUNK
    n_grp = n_ch // NBUF
    scale = math.sqrt(d)
    n_col = d // LANES
    row_unroll = 4

    mesh = plsc.VectorSubcoreMesh(core_axis_name="c", subcore_axis_name="s")

    @functools.partial(
        pl.kernel,
        mesh=mesh,
        compiler_params=pltpu.CompilerParams(use_tc_tiling_on_sc=False),
        out_type=jax.ShapeDtypeStruct((n_idx, OUTW), jnp.float32),
        scratch_types=[
            pltpu.VMEM((n_ch, CHUNK), jnp.int32),
            *[pltpu.VMEM((CHUNK, d), jnp.float32) for _ in range(NBUF)],
            *[pltpu.VMEM((CHUNK, OUTW), jnp.float32) for _ in range(NBUF)],
            *[pltpu.SemaphoreType.DMA for _ in range(2 * NBUF)],
        ],
    )
    def run(tok_hbm, table_hbm, out_hbm, idx_v, *rest):
        gbufs = rest[0:NBUF]
        sbufs = rest[NBUF:2 * NBUF]
        gsems = rest[2 * NBUF:3 * NBUF]
        ssems = rest[3 * NBUF:4 * NBUF]

        wid = lax.axis_index("s") * NC + lax.axis_index("c")
        base = wid * per_w

        pltpu.sync_copy(tok_hbm.at[wid], idx_v)

        def start_gather(j, b):
            pltpu.async_copy(table_hbm.at[idx_v.at[j]], gbufs[b], gsems[b])

        def wait_gather(b):
            pltpu.make_async_copy(
                table_hbm.at[pl.ds(0, CHUNK)], gbufs[b], gsems[b]).wait()

        def start_store(j, b):
            pltpu.async_copy(
                sbufs[b], out_hbm.at[pl.ds(base + j * CHUNK, CHUNK)], ssems[b])

        def wait_store(b):
            pltpu.make_async_copy(
                sbufs[b], out_hbm.at[pl.ds(0, CHUNK)], ssems[b]).wait()

        def scale_buf(b):
            gb, sb = gbufs[b], sbufs[b]

            def body(i, _):
                r0 = i * row_unroll
                for u in range(row_unroll):
                    for c in range(n_col):
                        sb[r0 + u, pl.ds(c * LANES, LANES)] = (
                            gb[r0 + u, pl.ds(c * LANES, LANES)] * scale)
                return 0

            lax.fori_loop(0, CHUNK // row_unroll, body, 0)

        for b in range(NBUF):
            start_gather(b, b)

        for b in range(NBUF):
            wait_gather(b)
            scale_buf(b)
            start_store(b, b)
            start_gather(b + NBUF, b)

        def group_body(g, _):
            for b in range(NBUF):
                j = g * NBUF + b
                wait_gather(b)
                wait_store(b)
                scale_buf(b)
                start_store(j, b)
                start_gather(j + NBUF, b)
            return 0

        lax.fori_loop(1, n_grp - 1, group_body, 0)

        for b in range(NBUF):
            j = (n_grp - 1) * NBUF + b
            wait_gather(b)
            wait_store(b)
            scale_buf(b)
            start_store(j, b)

        for b in range(NBUF):
            wait_store(b)

    return run


def kernel(tokens, embedding):
    b, l = tokens.shape
    vocab, d = embedding.shape
    n_idx = b * l
    assert n_idx % (NW * CHUNK * NBUF) == 0 and d % LANES == 0
    tok = tokens.reshape(NW, n_idx // (NW * CHUNK), CHUNK).astype(jnp.int32)
    out = _build(n_idx, vocab, d)(tok, embedding)
    res = out[:, :d].reshape(b, l, d)
    return with_layout_constraint(res, Layout(major_to_minor=(0, 1, 2)))


# speedup vs baseline: 1.6669x; 1.4314x over previous
import functools
import math

import jax
import jax.numpy as jnp
from jax import lax
from jax.experimental import pallas as pl
from jax.experimental.layout import Format, Layout, with_layout_constraint
from jax.experimental.pallas import tpu as pltpu
from jax.experimental.pallas import tpu_sc as plsc

LANES = 16
NC, NS = 2, 16
NW = NC * NS
CHUNK = 128
NBUF = 4
OUTW = 128


def _build(n_idx, vocab, d):
    per_w = n_idx // NW
    n_ch = per_w // CHUNK
    n_grp = n_ch // NBUF
    scale = math.sqrt(d)
    n_col = d // LANES
    row_unroll = 4

    mesh = plsc.VectorSubcoreMesh(core_axis_name="c", subcore_axis_name="s")

    @functools.partial(
        pl.kernel,
        mesh=mesh,
        compiler_params=pltpu.CompilerParams(use_tc_tiling_on_sc=False),
        out_type=jax.ShapeDtypeStruct((n_idx, OUTW), jnp.float32),
        scratch_types=[
            pltpu.VMEM((n_ch, CHUNK), jnp.int32),
            *[pltpu.VMEM((CHUNK, d), jnp.float32) for _ in range(NBUF)],
            *[pltpu.SemaphoreType.DMA for _ in range(2 * NBUF)],
        ],
    )
    def run(tok_hbm, table_hbm, out_hbm, idx_v, *rest):
        gbufs = rest[0:NBUF]
        gsems = rest[NBUF:2 * NBUF]
        ssems = rest[2 * NBUF:3 * NBUF]

        wid = lax.axis_index("s") * NC + lax.axis_index("c")
        base = wid * per_w

        pltpu.sync_copy(tok_hbm.at[wid], idx_v)

        def start_gather(j, b):
            pltpu.async_copy(table_hbm.at[idx_v.at[j]], gbufs[b], gsems[b])

        def wait_gather(b):
            pltpu.make_async_copy(
                table_hbm.at[pl.ds(0, CHUNK)], gbufs[b], gsems[b]).wait()

        def start_store(j, b):
            pltpu.async_copy(
                gbufs[b],
                out_hbm.at[pl.ds(base + j * CHUNK, CHUNK), pl.ds(0, d)],
                ssems[b])

        def wait_store(b):
            pltpu.make_async_copy(
                gbufs[b], out_hbm.at[pl.ds(0, CHUNK), pl.ds(0, d)],
                ssems[b]).wait()

        def scale_buf(b):
            gb = gbufs[b]

            def body(i, _):
                r0 = i * row_unroll
                for u in range(row_unroll):
                    for c in range(n_col):
                        gb[r0 + u, pl.ds(c * LANES, LANES)] = (
                            gb[r0 + u, pl.ds(c * LANES, LANES)] * scale)
                return 0

            lax.fori_loop(0, CHUNK // row_unroll, body, 0)

        for b in range(NBUF):
            start_gather(b, b)

        for b in range(NBUF):
            wait_gather(b)
            scale_buf(b)
            start_store(b, b)
            start_gather(b + NBUF, b)

        def group_body(g, _):
            for b in range(NBUF):
                j = g * NBUF + b
                wait_gather(b)
                wait_store(b)
                scale_buf(b)
                start_store(j, b)
                start_gather(j + NBUF, b)
            return 0

        lax.fori_loop(1, n_grp - 1, group_body, 0)

        for b in range(NBUF):
            j = (n_grp - 1) * NBUF + b
            wait_gather(b)
            wait_store(b)
            scale_buf(b)
            start_store(j, b)

        for b in range(NBUF):
            wait_store(b)

    return run


def kernel(tokens, embedding):
    b, l = tokens.shape
    vocab, d = embedding.shape
    n_idx = b * l
    assert n_idx % (NW * CHUNK * NBUF) == 0 and d % LANES == 0
    tok = tokens.reshape(NW, n_idx // (NW * CHUNK), CHUNK).astype(jnp.int32)
    out = _build(n_idx, vocab, d)(tok, embedding)
    res = out[:, :d].reshape(b, l, d)
    return with_layout_constraint(res, Layout(major_to_minor=(0, 1, 2)))
